# baseline (speedup 1.0000x reference)
import functools

import jax
import jax.numpy as jnp
from jax import lax
from jax.experimental import pallas as pl
from jax.experimental.pallas import tpu as pltpu
from jax.experimental.pallas import tpu_sc as plsc

_D = 128
_C = 128
_NBUF = 5


def kernel(rule_ids, table):
    B, T = rule_ids.shape
    N = B * T
    ids = rule_ids.reshape(N)

    info = plsc.get_sparse_core_info()
    nc, ns = info.num_cores, info.num_subcores
    nw = nc * ns
    R = 16
    table_pad = jnp.pad(table, ((0, R - table.shape[0]), (0, 0)))
    per_w = N // nw
    gc = _NBUF * _C
    n_grp = per_w // gc
    assert per_w * nw == N and n_grp * gc == per_w

    mesh = plsc.VectorSubcoreMesh(core_axis_name="c", subcore_axis_name="s")

    @functools.partial(
        pl.kernel,
        out_type=jax.ShapeDtypeStruct((N, _D), jnp.float32),
        mesh=mesh,
        scratch_types=[
            pltpu.VMEM((2, gc), jnp.int32),
            pltpu.VMEM((_NBUF, _C, _D), jnp.float32),
            pltpu.VMEM((R, _D), jnp.float32),
            pltpu.VMEM_SHARED((R, _D), jnp.float32),
            [pltpu.SemaphoreType.DMA] * _NBUF,
            [pltpu.SemaphoreType.DMA] * _NBUF,
            [pltpu.SemaphoreType.DMA] * 2,
        ],
    )
    def run(ids_hbm, table_hbm, out_hbm, idx_v, rows, tbl_v, tbl_sh,
            gsems, osems, isems):
        wid = lax.axis_index("s") * nc + lax.axis_index("c")
        base = wid * per_w

        pltpu.sync_copy(table_hbm, tbl_v)

        @pl.when(lax.axis_index("s") == 0)
        def _publish():
            pltpu.sync_copy(tbl_v, tbl_sh)

        plsc.subcore_barrier()
        my_table = tbl_sh

        def ids_copy(g, p):
            return pltpu.make_async_copy(
                ids_hbm.at[pl.ds(base + g * gc, gc)], idx_v.at[p], isems[p])

        ids_copy(0, 0).start()

        def grp_pair(s, carry):
            for p in range(2):
                g = 2 * s + p
                goff = base + g * gc
                ids_copy(g, p).wait()
                g_next = jnp.minimum(g + 1, n_grp - 1)
                ids_copy(g_next, 1 - p).start()
                gathers = []
                for b in range(_NBUF):
                    def _wait_prev_store(b=b, off_prev=goff - gc + b * _C):
                        pltpu.make_async_copy(
                            rows.at[b], out_hbm.at[pl.ds(off_prev, _C)],
                            osems[b]).wait()

                    if p == 0:
                        pl.when(s > 0)(_wait_prev_store)
                    else:
                        _wait_prev_store()

                    gathers.append(pltpu.async_copy(
                        my_table.at[idx_v.at[p, pl.ds(b * _C, _C)]],
                        rows.at[b], gsems[b]))
                for b in range(_NBUF):
                    gathers[b].wait()
                    pltpu.async_copy(
                        rows.at[b], out_hbm.at[pl.ds(goff + b * _C, _C)],
                        osems[b])
            return carry

        lax.fori_loop(0, n_grp // 2, grp_pair, 0)
        ids_copy(n_grp - 1, 0).wait()
        for b in range(_NBUF):
            pltpu.make_async_copy(
                rows.at[b], out_hbm.at[pl.ds(base + (n_grp - 1) * gc + b * _C, _C)],
                osems[b]).wait()

    out = run(ids, table_pad)
    return out.reshape(B, T, _D)


# speedup vs baseline: 1.0074x; 1.0074x over previous
import functools

import jax
import jax.numpy as jnp
from jax import lax
from jax.experimental import pallas as pl
from jax.experimental.pallas import tpu as pltpu
from jax.experimental.pallas import tpu_sc as plsc

_D = 128
_C = 128
_NBUF = 4


def kernel(rule_ids, table):
    B, T = rule_ids.shape
    N = B * T
    ids = rule_ids.reshape(N)

    info = plsc.get_sparse_core_info()
    nc, ns = info.num_cores, info.num_subcores
    nw = nc * ns
    R = 16
    table_pad = jnp.pad(table, ((0, R - table.shape[0]), (0, 0)))
    per_w = N // nw
    gc = _NBUF * _C
    n_grp = per_w // gc
    assert per_w * nw == N and n_grp * gc == per_w

    mesh = plsc.VectorSubcoreMesh(core_axis_name="c", subcore_axis_name="s")

    @functools.partial(
        pl.kernel,
        out_type=jax.ShapeDtypeStruct((N, _D), jnp.float32),
        mesh=mesh,
        scratch_types=[
            pltpu.VMEM((2, gc), jnp.int32),
            pltpu.VMEM((_NBUF, _C, _D), jnp.float32),
            pltpu.VMEM((R, _D), jnp.float32),
            pltpu.VMEM_SHARED((R, _D), jnp.float32),
            [pltpu.SemaphoreType.DMA] * _NBUF,
            [pltpu.SemaphoreType.DMA] * _NBUF,
            [pltpu.SemaphoreType.DMA] * 2,
        ],
    )
    def run(ids_hbm, table_hbm, out_hbm, idx_v, rows, tbl_v, tbl_sh,
            gsems, osems, isems):
        wid = lax.axis_index("s") * nc + lax.axis_index("c")
        base = wid * per_w

        pltpu.sync_copy(table_hbm, tbl_v)

        @pl.when(lax.axis_index("s") == 0)
        def _publish():
            pltpu.sync_copy(tbl_v, tbl_sh)

        plsc.subcore_barrier()
        my_table = tbl_sh

        def ids_copy(g, p):
            return pltpu.make_async_copy(
                ids_hbm.at[pl.ds(base + g * gc, gc)], idx_v.at[p], isems[p])

        ids_copy(0, 0).start()

        def grp_pair(s, carry):
            for p in range(2):
                g = 2 * s + p
                goff = base + g * gc
                ids_copy(g, p).wait()
                g_next = jnp.minimum(g + 1, n_grp - 1)
                ids_copy(g_next, 1 - p).start()
                gathers = []
                for b in range(_NBUF):
                    def _wait_prev_store(b=b, off_prev=goff - gc + b * _C):
                        pltpu.make_async_copy(
                            rows.at[b], out_hbm.at[pl.ds(off_prev, _C)],
                            osems[b]).wait()

                    if p == 0:
                        pl.when(s > 0)(_wait_prev_store)
                    else:
                        _wait_prev_store()

                    gathers.append(pltpu.async_copy(
                        my_table.at[idx_v.at[p, pl.ds(b * _C, _C)]],
                        rows.at[b], gsems[b]))
                for b in range(_NBUF):
                    gathers[b].wait()
                    pltpu.async_copy(
                        rows.at[b], out_hbm.at[pl.ds(goff + b * _C, _C)],
                        osems[b])
            return carry

        lax.fori_loop(0, n_grp // 2, grp_pair, 0)
        ids_copy(n_grp - 1, 0).wait()
        for b in range(_NBUF):
            pltpu.make_async_copy(
                rows.at[b], out_hbm.at[pl.ds(base + (n_grp - 1) * gc + b * _C, _C)],
                osems[b]).wait()

    out = run(ids, table_pad)
    return out.reshape(B, T, _D)
